# Initial kernel scaffold; baseline (speedup 1.0000x reference)
import functools

import jax
import jax.numpy as jnp
from jax import lax
from jax.experimental import pallas as pl
from jax.experimental.pallas import tpu as pltpu
from jax.experimental.pallas import tpu_sc as plsc

L = 16


def _sc_info():
    try:
        info = plsc.get_sparse_core_info()
        return info.num_cores, info.num_subcores
    except Exception:
        return 2, 16


@functools.lru_cache(maxsize=None)
def _make_sc_call(B, S, V, D, P):
    NC, NS = _sc_info()
    NW = NC * NS
    assert B % NW == 0
    RPW = B // NW
    S0 = 128
    S1 = S - S0
    NCH0 = S0 // L
    NCH1 = (S1 + L - 1) // L

    mesh = plsc.VectorSubcoreMesh(core_axis_name="c", subcore_axis_name="s")

    @functools.partial(
        pl.kernel,
        out_type=jax.ShapeDtypeStruct((B * S, D), jnp.float32),
        mesh=mesh,
        scratch_types=[
            pltpu.VMEM((RPW * 2, S0), jnp.int32),
            pltpu.VMEM((2, S0), jnp.int32),
            pltpu.VMEM((S, D), jnp.float32),
            pltpu.SemaphoreType.DMA,
        ],
    )
    def sc_embed(xp_hbm, tok_hbm, pos_hbm, out_hbm, idx_all, posidx, rows_v, sem):
        wid = lax.axis_index("s") * NC + lax.axis_index("c")
        pltpu.sync_copy(xp_hbm.at[pl.ds(wid * (RPW * 2), RPW * 2)], idx_all)

        def row_body(i, dummy):
            carry = jnp.int32(0)
            for part, nch in ((0, NCH0), (1, NCH1)):
                r = 2 * i + part
                for c in range(nch):
                    tok = idx_all[r, pl.ds(c * L, L)]
                    nz = jnp.where(tok != 0, 1, 0).astype(jnp.int32)
                    cs = plsc.cumsum(nz) + carry
                    posidx[part, pl.ds(c * L, L)] = jnp.where(tok == 0, 0, cs)
                    carry = carry + jnp.sum(nz)
            pltpu.sync_copy(pos_hbm.at[posidx.at[0]], rows_v.at[pl.ds(0, S0)])
            pltpu.sync_copy(pos_hbm.at[posidx.at[1, pl.ds(0, S1)]],
                            rows_v.at[pl.ds(S0, S1)])
            d0 = pltpu.async_copy(tok_hbm.at[idx_all.at[2 * i]],
                                  rows_v.at[pl.ds(0, S0)], sem, add=True)
            d1 = pltpu.async_copy(tok_hbm.at[idx_all.at[2 * i + 1, pl.ds(0, S1)]],
                                  rows_v.at[pl.ds(S0, S1)], sem, add=True)
            d0.wait()
            d1.wait()
            tbase = (wid * RPW + i) * S
            pltpu.sync_copy(rows_v, out_hbm.at[pl.ds(tbase, S)])
            return dummy

        lax.fori_loop(0, RPW, row_body, jnp.int32(0))

    return sc_embed


def kernel(x, tok_table, pos_table):
    B, S = x.shape
    V, D = tok_table.shape
    P = pos_table.shape[0]
    xp = jnp.pad(x, ((0, 0), (0, 256 - S))).reshape(B * 2, 128)
    out_flat = _make_sc_call(B, S, V, D, P)(xp, tok_table, pos_table)
    return out_flat.reshape(B, S, D), (x == 0)


# speedup vs baseline: 2.0251x; 2.0251x over previous
import functools

import jax
import jax.numpy as jnp
from jax import lax
from jax.experimental import pallas as pl
from jax.experimental.pallas import tpu as pltpu
from jax.experimental.pallas import tpu_sc as plsc

L = 16


def _cumsum16(v):
    iota = lax.iota(jnp.int32, L)
    for k in (1, 2, 4, 8):
        idx = jnp.maximum(iota - k, 0)
        v = v + jnp.where(iota >= k, v[idx], 0)
    return v


def _sc_info():
    try:
        info = plsc.get_sparse_core_info()
        return info.num_cores, info.num_subcores
    except Exception:
        return 2, 16


@functools.lru_cache(maxsize=None)
def _make_sc_call(B, S, V, D, P):
    NC, NS = _sc_info()
    NW = NC * NS
    assert B % NW == 0
    RPW = B // NW
    S0 = 128
    S1 = S - S0
    NCH0 = S0 // L
    NCH1 = (S1 + L - 1) // L

    mesh = plsc.VectorSubcoreMesh(core_axis_name="c", subcore_axis_name="s")

    @functools.partial(
        pl.kernel,
        out_type=jax.ShapeDtypeStruct((B * S, D), jnp.float32),
        mesh=mesh,
        scratch_types=[
            pltpu.VMEM((RPW * 2, S0), jnp.int32),
            pltpu.VMEM((2, S0), jnp.int32),
            pltpu.VMEM((S, D), jnp.float32),
            pltpu.SemaphoreType.DMA,
        ],
        compiler_params=pltpu.CompilerParams(use_tc_tiling_on_sc=False),
    )
    def sc_embed(xp_hbm, tok_hbm, pos_hbm, out_hbm, idx_all, posidx, rows_v, sem):
        wid = lax.axis_index("s") * NC + lax.axis_index("c")
        pltpu.sync_copy(xp_hbm.at[pl.ds(wid * (RPW * 2), RPW * 2)], idx_all)

        def row_body(i, dummy):
            carry = jnp.int32(0)
            for part, nch in ((0, NCH0), (1, NCH1)):
                r = 2 * i + part
                for c in range(nch):
                    tok = idx_all[r, pl.ds(c * L, L)]
                    nz = jnp.where(tok != 0, 1, 0).astype(jnp.int32)
                    cs = _cumsum16(nz) + carry
                    posidx[part, pl.ds(c * L, L)] = jnp.where(tok == 0, 0, cs)
                    carry = cs[L - 1]
            pltpu.sync_copy(pos_hbm.at[posidx.at[0]], rows_v.at[pl.ds(0, S0)])
            pltpu.sync_copy(pos_hbm.at[posidx.at[1, pl.ds(0, S1)]],
                            rows_v.at[pl.ds(S0, S1)])
            d0 = pltpu.async_copy(tok_hbm.at[idx_all.at[2 * i]],
                                  rows_v.at[pl.ds(0, S0)], sem, add=True)
            d1 = pltpu.async_copy(tok_hbm.at[idx_all.at[2 * i + 1, pl.ds(0, S1)]],
                                  rows_v.at[pl.ds(S0, S1)], sem, add=True)
            d0.wait()
            d1.wait()
            tbase = (wid * RPW + i) * S
            pltpu.sync_copy(rows_v, out_hbm.at[pl.ds(tbase, S)])
            return dummy

        lax.fori_loop(0, RPW, row_body, jnp.int32(0))

    return sc_embed


def kernel(x, tok_table, pos_table):
    B, S = x.shape
    V, D = tok_table.shape
    P = pos_table.shape[0]
    xp = jnp.pad(x, ((0, 0), (0, 256 - S))).reshape(B * 2, 128)
    out_flat = _make_sc_call(B, S, V, D, P)(xp, tok_table, pos_table)
    return out_flat.reshape(B, S, D), (x == 0)
